# baseline (speedup 1.0000x reference)
import functools

import jax
import jax.numpy as jnp
import numpy as np
from jax import lax
from jax.experimental import pallas as pl
from jax.experimental.pallas import tpu as pltpu
from jax.experimental.pallas import tpu_sc as plsc

_FIELD_DIMS = np.array([10000] * 39, dtype=np.int64)
_SEL = np.hstack((_FIELD_DIMS[:3], _FIELD_DIMS[4:8], _FIELD_DIMS[10:15],
                  _FIELD_DIMS[17:19], _FIELD_DIMS[21:24], _FIELD_DIMS[26:]))
_OFFSETS = np.array((0, *np.cumsum(_SEL)[:-1]), dtype=np.int32)
_SELIDS = np.array([*range(0, 3), *range(4, 8), *range(10, 15),
                    *range(17, 19), *range(21, 24), *range(26, 39)],
                   dtype=np.int32)

B = 4096
F = 30
FP = 32
D = 64
DP = 128
V = 300000
NC, NS, L = 2, 16, 16
NW = NC * NS
BW = B // NW
ROWS = 128
C = ROWS // FP
NCHUNK = BW // C
XROWS = 40

NBLK = V // DP
TAIL0 = NBLK * DP
NI = -(-NBLK // NW)

_SELP = np.concatenate([_SELIDS, [0, 1]]).astype(np.int32)
_OFFP = np.concatenate([_OFFSETS, [280000, 290000]]).astype(np.int32)
_SELOFF = np.concatenate([_SELP, _OFFP, np.zeros(64, np.int32)])

_PARAMS = pltpu.CompilerParams(needs_layout_passes=False,
                               use_tc_tiling_on_sc=True)


def _mesh():
  return plsc.VectorSubcoreMesh(core_axis_name="c", subcore_axis_name="s",
                                num_cores=NC, num_subcores=NS)


def _build_transpose(interpret=False):
  @functools.partial(
      pl.kernel,
      out_type=jax.ShapeDtypeStruct((V, DP), jnp.float32),
      mesh=_mesh(),
      interpret=interpret,
      compiler_params=_PARAMS,
      scratch_types=[
          pltpu.VMEM((D, DP), jnp.float32),
          pltpu.VMEM((D, DP), jnp.float32),
          pltpu.VMEM((DP, DP), jnp.float32),
          pltpu.VMEM((DP, DP), jnp.float32),
          pltpu.SemaphoreType.DMA,
          pltpu.SemaphoreType.DMA,
          pltpu.SemaphoreType.DMA,
          pltpu.SemaphoreType.DMA,
      ],
  )
  def t_kernel(tt_hbm, tail_hbm, out_hbm, in0, in1, ot0, ot1,
               is0, is1, os0, os1):
    wid = lax.axis_index("s") * NC + lax.axis_index("c")
    lanes = lax.iota(jnp.int32, L)

    def c0_of(i):
      ct = jnp.minimum(wid + NW * i, NBLK - 1)
      return pl.multiple_of(ct * DP, DP)

    def in_start(i, buf, sem):
      pltpu.async_copy(tt_hbm.at[:, pl.ds(c0_of(i), DP)], buf, sem)

    def in_wait(i, buf, sem):
      pltpu.make_async_copy(tt_hbm.at[:, pl.ds(c0_of(i), DP)], buf, sem).wait()

    def out_start(i, buf, sem):
      pltpu.async_copy(buf, out_hbm.at[pl.ds(c0_of(i), DP)], sem)

    def out_wait(i, buf, sem):
      pltpu.make_async_copy(buf, out_hbm.at[pl.ds(c0_of(i), DP)], sem).wait()

    def transpose(in_ref, out_ref):
      def rbody(r, _):
        cvec = jnp.zeros((L,), jnp.int32) + r
        for k in range(D // L):
          v = plsc.load_gather(in_ref, [lanes + k * L, cvec])
          out_ref[r, pl.ds(k * L, L)] = v
        return 0

      lax.fori_loop(0, DP, rbody, 0)

    in_start(0, in0, is0)
    in_start(1, in1, is1)

    def pipe(j, _):
      i0 = 2 * j
      in_wait(i0, in0, is0)

      @pl.when(j > 0)
      def _():
        out_wait(i0 - 2, ot0, os0)

      transpose(in0, ot0)
      out_start(i0, ot0, os0)

      @pl.when(j < NI // 2 - 1)
      def _():
        in_start(i0 + 2, in0, is0)

      in_wait(i0 + 1, in1, is1)

      @pl.when(j > 0)
      def _():
        out_wait(i0 - 1, ot1, os1)

      transpose(in1, ot1)
      out_start(i0 + 1, ot1, os1)

      @pl.when(j < NI // 2 - 1)
      def _():
        in_start(i0 + 3, in1, is1)

      return 0

    @pl.when(wid == 0)
    def _():
      pltpu.sync_copy(tail_hbm, ot0.at[pl.ds(0, V - TAIL0)])
      pltpu.sync_copy(ot0.at[pl.ds(0, V - TAIL0)],
                      out_hbm.at[pl.ds(TAIL0, V - TAIL0)])

    lax.fori_loop(0, NI // 2, pipe, 0)
    out_wait(NI - 2, ot0, os0)
    out_wait(NI - 1, ot1, os1)

  return t_kernel


def _build_fm(interpret=False):
  @functools.partial(
      pl.kernel,
      out_type=jax.ShapeDtypeStruct((B,), jnp.float32),
      mesh=_mesh(),
      interpret=interpret,
      compiler_params=_PARAMS,
      scratch_types=[
          pltpu.VMEM((128,), jnp.int32),
          pltpu.VMEM((XROWS, BW), jnp.int32),
          pltpu.VMEM((NCHUNK, ROWS), jnp.int32),
          pltpu.VMEM((2, ROWS, DP), jnp.float32),
          pltpu.VMEM((BW,), jnp.float32),
          pltpu.VMEM((128,), jnp.float32),
          pltpu.SemaphoreType.DMA,
          pltpu.SemaphoreType.DMA,
      ],
  )
  def fm_kernel(seloff_hbm, x_hbm, table_hbm, bias_hbm, out_hbm,
                seloff_v, x_v, idx_v, rows_v, out_v, bias_v, sem0, sem1):
    wid = lax.axis_index("s") * NC + lax.axis_index("c")
    sems = (sem0, sem1)

    pltpu.sync_copy(seloff_hbm, seloff_v)
    pltpu.sync_copy(x_hbm.at[:, pl.ds(wid * BW, BW)], x_v)
    pltpu.sync_copy(bias_hbm, bias_v)

    selv = [seloff_v[pl.ds(h * L, L)] for h in range(FP // L)]
    offv = [seloff_v[pl.ds(FP + h * L, L)] for h in range(FP // L)]

    def build_chunk(c, _):
      for bb in range(C):
        bvec = jnp.zeros((L,), jnp.int32) + (c * C + bb)
        for h in range(FP // L):
          v = plsc.load_gather(x_v, [selv[h], bvec]) + offv[h]
          idx_v[c, pl.ds(bb * FP + h * L, L)] = v
      return 0

    lax.fori_loop(0, NCHUNK, build_chunk, 0)

    def gather_start(c, buf):
      pltpu.async_copy(table_hbm.at[idx_v.at[c]], rows_v.at[buf], sems[buf])

    def gather_wait(c, buf):
      pltpu.make_async_copy(table_hbm.at[idx_v.at[c]], rows_v.at[buf],
                            sems[buf]).wait()

    lanes = lax.iota(jnp.int32, L)

    def compute_chunk(c, buf, tvec):
      for bb in range(C):
        zero = jnp.zeros((L,), jnp.float32)

        def fbody(f, carry, _bb=bb):
          s0, s1, s2, s3, q = carry
          j = _bb * FP + f
          r0 = rows_v[buf, j, pl.ds(0, L)]
          r1 = rows_v[buf, j, pl.ds(L, L)]
          r2 = rows_v[buf, j, pl.ds(2 * L, L)]
          r3 = rows_v[buf, j, pl.ds(3 * L, L)]
          return (s0 + r0, s1 + r1, s2 + r2, s3 + r3,
                  q + r0 * r0 + r1 * r1 + r2 * r2 + r3 * r3)

        s0, s1, s2, s3, q = lax.fori_loop(0, F, fbody, (zero,) * 5)
        lin = jnp.sum(s0 + s1 + s2 + s3)
        sq = jnp.sum(s0 * s0 + s1 * s1 + s2 * s2 + s3 * s3)
        qs = jnp.sum(q)
        t = lin + 0.5 * (sq - qs)
        lane = (c * C + bb) % L
        tvec = jnp.where(lanes == lane, t, tvec)
      return tvec

    gather_start(0, 0)

    def pipe_body(i, tvec):
      c0 = 2 * i
      gather_start(c0 + 1, 1)
      gather_wait(c0, 0)
      tvec = compute_chunk(c0, 0, tvec)

      @pl.when(i < NCHUNK // 2 - 1)
      def _():
        gather_start(c0 + 2, 0)

      gather_wait(c0 + 1, 1)
      tvec = compute_chunk(c0 + 1, 1, tvec)

      @pl.when(i % 2 == 1)
      def _():
        out_v[pl.ds((i // 2) * L, L)] = tvec

      return tvec

    lax.fori_loop(0, NCHUNK // 2, pipe_body, jnp.zeros((L,), jnp.float32))

    bias_vec = bias_v[pl.ds(0, L)]
    for k in range(BW // L):
      t = out_v[pl.ds(k * L, L)] + bias_vec
      out_v[pl.ds(k * L, L)] = 1.0 / (1.0 + jnp.exp(-t))

    pltpu.sync_copy(out_v, out_hbm.at[pl.ds(wid * BW, BW)])

  return fm_kernel


_CACHE = {}


def _get(name, builder):
  if name not in _CACHE:
    _CACHE[name] = builder()
  return _CACHE[name]


@jax.jit
def kernel(x, additional, emb_table, bias):
  del additional
  xt = jnp.concatenate(
      (x.astype(jnp.int32).T, jnp.zeros((XROWS - 39, B), jnp.int32)), axis=0)
  bias128 = jnp.broadcast_to(bias.astype(jnp.float32), (128,))
  tail128 = jnp.pad(emb_table[TAIL0:, :], ((0, 0), (0, DP - D)))
  t128 = _get("t", _build_transpose)(emb_table.T, tail128)
  return _get("fm", _build_fm)(jnp.asarray(_SELOFF), xt, t128, bias128)


# speedup vs baseline: 3.4655x; 3.4655x over previous
import functools

import jax
import jax.numpy as jnp
import numpy as np
from jax import lax
from jax.experimental import pallas as pl
from jax.experimental.pallas import tpu as pltpu
from jax.experimental.pallas import tpu_sc as plsc

_FIELD_DIMS = np.array([10000] * 39, dtype=np.int64)
_SEL = np.hstack((_FIELD_DIMS[:3], _FIELD_DIMS[4:8], _FIELD_DIMS[10:15],
                  _FIELD_DIMS[17:19], _FIELD_DIMS[21:24], _FIELD_DIMS[26:]))
_OFFSETS = np.array((0, *np.cumsum(_SEL)[:-1]), dtype=np.int32)
_SELIDS = np.array([*range(0, 3), *range(4, 8), *range(10, 15),
                    *range(17, 19), *range(21, 24), *range(26, 39)],
                   dtype=np.int32)

B = 4096
F = 30
FP = 32
D = 64
DP = 128
V = 300000
NC, NS, L = 2, 16, 16
NW = NC * NS
BW = B // NW
ROWS = 128
C = ROWS // FP
NCHUNK = BW // C
XROWS = 40

NBLK = V // DP
TAIL0 = NBLK * DP
NI = -(-NBLK // NW)

_SELP = np.concatenate([_SELIDS, [0, 1]]).astype(np.int32)
_OFFP = np.concatenate([_OFFSETS, [280000, 290000]]).astype(np.int32)
_SELOFF = np.concatenate([_SELP, _OFFP, np.zeros(64, np.int32)])

_PARAMS = pltpu.CompilerParams(needs_layout_passes=False,
                               use_tc_tiling_on_sc=True)


def _mesh():
  return plsc.VectorSubcoreMesh(core_axis_name="c", subcore_axis_name="s",
                                num_cores=NC, num_subcores=NS)


def _build_transpose(interpret=False):
  @functools.partial(
      pl.kernel,
      out_type=jax.ShapeDtypeStruct((V, DP), jnp.float32),
      mesh=_mesh(),
      interpret=interpret,
      compiler_params=_PARAMS,
      scratch_types=[
          pltpu.VMEM((D, DP), jnp.float32),
          pltpu.VMEM((D, DP), jnp.float32),
          pltpu.VMEM((DP, DP), jnp.float32),
          pltpu.VMEM((DP, DP), jnp.float32),
          pltpu.SemaphoreType.DMA,
          pltpu.SemaphoreType.DMA,
          pltpu.SemaphoreType.DMA,
          pltpu.SemaphoreType.DMA,
      ],
  )
  def t_kernel(tt_hbm, tail_hbm, out_hbm, in0, in1, ot0, ot1,
               is0, is1, os0, os1):
    wid = lax.axis_index("s") * NC + lax.axis_index("c")
    lanes = lax.iota(jnp.int32, L)

    def c0_of(i):
      ct = jnp.minimum(wid + NW * i, NBLK - 1)
      return pl.multiple_of(ct * DP, DP)

    def in_start(i, buf, sem):
      pltpu.async_copy(tt_hbm.at[:, pl.ds(c0_of(i), DP)], buf, sem)

    def in_wait(i, buf, sem):
      pltpu.make_async_copy(tt_hbm.at[:, pl.ds(c0_of(i), DP)], buf, sem).wait()

    def out_start(i, buf, sem):
      pltpu.async_copy(buf, out_hbm.at[pl.ds(c0_of(i), DP)], sem)

    def out_wait(i, buf, sem):
      pltpu.make_async_copy(buf, out_hbm.at[pl.ds(c0_of(i), DP)], sem).wait()

    dvecs = [lanes + k * L for k in range(D // L)]

    def transpose(in_ref, out_ref):
      @functools.partial(plsc.parallel_loop, 0, DP, unroll=8)
      def _(r):
        cvec = jnp.zeros((L,), jnp.int32) + r
        for k in range(D // L):
          v = plsc.load_gather(in_ref, [dvecs[k], cvec])
          out_ref[r, pl.ds(k * L, L)] = v

    in_start(0, in0, is0)
    in_start(1, in1, is1)

    def pipe(j, _):
      i0 = 2 * j
      in_wait(i0, in0, is0)

      @pl.when(j > 0)
      def _():
        out_wait(i0 - 2, ot0, os0)

      transpose(in0, ot0)
      out_start(i0, ot0, os0)

      @pl.when(j < NI // 2 - 1)
      def _():
        in_start(i0 + 2, in0, is0)

      in_wait(i0 + 1, in1, is1)

      @pl.when(j > 0)
      def _():
        out_wait(i0 - 1, ot1, os1)

      transpose(in1, ot1)
      out_start(i0 + 1, ot1, os1)

      @pl.when(j < NI // 2 - 1)
      def _():
        in_start(i0 + 3, in1, is1)

      return 0

    @pl.when(wid == 0)
    def _():
      pltpu.sync_copy(tail_hbm, ot0.at[pl.ds(0, V - TAIL0)])
      pltpu.sync_copy(ot0.at[pl.ds(0, V - TAIL0)],
                      out_hbm.at[pl.ds(TAIL0, V - TAIL0)])

    lax.fori_loop(0, NI // 2, pipe, 0)
    out_wait(NI - 2, ot0, os0)
    out_wait(NI - 1, ot1, os1)

  return t_kernel


def _build_fm(interpret=False):
  @functools.partial(
      pl.kernel,
      out_type=jax.ShapeDtypeStruct((B,), jnp.float32),
      mesh=_mesh(),
      interpret=interpret,
      compiler_params=_PARAMS,
      scratch_types=[
          pltpu.VMEM((128,), jnp.int32),
          pltpu.VMEM((XROWS, BW), jnp.int32),
          pltpu.VMEM((NCHUNK, ROWS), jnp.int32),
          pltpu.VMEM((2, ROWS, DP), jnp.float32),
          pltpu.VMEM((BW,), jnp.float32),
          pltpu.VMEM((128,), jnp.float32),
          pltpu.SemaphoreType.DMA,
          pltpu.SemaphoreType.DMA,
      ],
  )
  def fm_kernel(seloff_hbm, x_hbm, table_hbm, bias_hbm, out_hbm,
                seloff_v, x_v, idx_v, rows_v, out_v, bias_v, sem0, sem1):
    wid = lax.axis_index("s") * NC + lax.axis_index("c")
    sems = (sem0, sem1)

    pltpu.sync_copy(seloff_hbm, seloff_v)
    pltpu.sync_copy(x_hbm.at[:, pl.ds(wid * BW, BW)], x_v)
    pltpu.sync_copy(bias_hbm, bias_v)

    selv = [seloff_v[pl.ds(h * L, L)] for h in range(FP // L)]
    offv = [seloff_v[pl.ds(FP + h * L, L)] for h in range(FP // L)]

    def build_chunk(c, _):
      for bb in range(C):
        bvec = jnp.zeros((L,), jnp.int32) + (c * C + bb)
        for h in range(FP // L):
          v = plsc.load_gather(x_v, [selv[h], bvec]) + offv[h]
          idx_v[c, pl.ds(bb * FP + h * L, L)] = v
      return 0

    lax.fori_loop(0, NCHUNK, build_chunk, 0)

    def gather_start(c, buf):
      pltpu.async_copy(table_hbm.at[idx_v.at[c]], rows_v.at[buf], sems[buf])

    def gather_wait(c, buf):
      pltpu.make_async_copy(table_hbm.at[idx_v.at[c]], rows_v.at[buf],
                            sems[buf]).wait()

    lanes = lax.iota(jnp.int32, L)

    def compute_chunk(c, buf, tvec):
      for bb in range(C):
        zero = jnp.zeros((L,), jnp.float32)

        def fbody(f, carry, _bb=bb):
          s0, s1, s2, s3, q = carry
          j = _bb * FP + f
          r0 = rows_v[buf, j, pl.ds(0, L)]
          r1 = rows_v[buf, j, pl.ds(L, L)]
          r2 = rows_v[buf, j, pl.ds(2 * L, L)]
          r3 = rows_v[buf, j, pl.ds(3 * L, L)]
          return (s0 + r0, s1 + r1, s2 + r2, s3 + r3,
                  q + r0 * r0 + r1 * r1 + r2 * r2 + r3 * r3)

        s0, s1, s2, s3, q = lax.fori_loop(0, F, fbody, (zero,) * 5)
        lin = jnp.sum(s0 + s1 + s2 + s3)
        sq = jnp.sum(s0 * s0 + s1 * s1 + s2 * s2 + s3 * s3)
        qs = jnp.sum(q)
        t = lin + 0.5 * (sq - qs)
        lane = (c * C + bb) % L
        tvec = jnp.where(lanes == lane, t, tvec)
      return tvec

    gather_start(0, 0)

    def pipe_body(i, tvec):
      c0 = 2 * i
      gather_start(c0 + 1, 1)
      gather_wait(c0, 0)
      tvec = compute_chunk(c0, 0, tvec)

      @pl.when(i < NCHUNK // 2 - 1)
      def _():
        gather_start(c0 + 2, 0)

      gather_wait(c0 + 1, 1)
      tvec = compute_chunk(c0 + 1, 1, tvec)

      @pl.when(i % 2 == 1)
      def _():
        out_v[pl.ds((i // 2) * L, L)] = tvec

      return tvec

    lax.fori_loop(0, NCHUNK // 2, pipe_body, jnp.zeros((L,), jnp.float32))

    bias_vec = bias_v[pl.ds(0, L)]
    for k in range(BW // L):
      t = out_v[pl.ds(k * L, L)] + bias_vec
      out_v[pl.ds(k * L, L)] = 1.0 / (1.0 + jnp.exp(-t))

    pltpu.sync_copy(out_v, out_hbm.at[pl.ds(wid * BW, BW)])

  return fm_kernel


_CACHE = {}


def _get(name, builder):
  if name not in _CACHE:
    _CACHE[name] = builder()
  return _CACHE[name]


@jax.jit
def kernel(x, additional, emb_table, bias):
  del additional
  xt = jnp.concatenate(
      (x.astype(jnp.int32).T, jnp.zeros((XROWS - 39, B), jnp.int32)), axis=0)
  bias128 = jnp.broadcast_to(bias.astype(jnp.float32), (128,))
  tail128 = jnp.pad(emb_table[TAIL0:, :], ((0, 0), (0, DP - D)))
  t128 = _get("t", _build_transpose)(emb_table.T, tail128)
  return _get("fm", _build_fm)(jnp.asarray(_SELOFF), xt, t128, bias128)
